# baseline (speedup 1.0000x reference)
import functools

import jax
import jax.numpy as jnp
from jax import lax
from jax.experimental import pallas as pl
from jax.experimental.pallas import tpu as pltpu
from jax.experimental.pallas import tpu_sc as plsc

N = 50000
E = 800000
NW = 32
NB = 64
NG = NB // 16
R = 784
RT = 792
NPAD = NB * R
EPW = E // NW
PAD_EPW = 25008
STG = 33152
GPACK = E + NW * NB * 128
BR = 2000
SINK = NPAD - 1
SINKLOC = RT - 1
CH = 1024
CQ = CH // 128

_mesh = plsc.VectorSubcoreMesh(core_axis_name="c", subcore_axis_name="s")
_sc_params = pltpu.CompilerParams(needs_layout_passes=False,
                                  use_tc_tiling_on_sc=False)


def _f32(*shape):
    return jax.ShapeDtypeStruct(shape, jnp.float32)


def _i32(*shape):
    return jax.ShapeDtypeStruct(shape, jnp.int32)


def _mo8(x):
    return pl.multiple_of(x, 8)


def _tc1_body(x_ref, wp_ref, bp_ref, ws_ref, p1_ref, xs1_ref):
    x = x_ref[...]
    p1_ref[...] = jnp.maximum(
        jnp.dot(x, wp_ref[...], preferred_element_type=jnp.float32) + bp_ref[...], 0.0)
    xs1_ref[...] = jnp.dot(x, ws_ref[...], preferred_element_type=jnp.float32)


def _tc2_body(xs1_ref, agg_ref, w1n_ref, b1_ref, w2p_ref, b2p_ref, w2s_ref,
              p2_ref, hs2_ref):
    h1 = jnp.maximum(
        xs1_ref[...]
        + jnp.dot(agg_ref[...], w1n_ref[...], preferred_element_type=jnp.float32)
        + b1_ref[...], 0.0)
    p2_ref[...] = jnp.maximum(
        jnp.dot(h1, w2p_ref[...], preferred_element_type=jnp.float32) + b2p_ref[...], 0.0)
    hs2_ref[...] = jnp.dot(h1, w2s_ref[...], preferred_element_type=jnp.float32)


def _tc3_body(hs2_ref, agg_ref, w2n_ref, b2_ref, h2_ref):
    h2_ref[...] = (hs2_ref[...]
                   + jnp.dot(agg_ref[...], w2n_ref[...], preferred_element_type=jnp.float32)
                   + b2_ref[...])


def _tc4_body(h2_ref, sums_ref, deg_ref, w3s_ref, w3n_ref, b3_ref, out_ref):
    agg = sums_ref[...] / jnp.maximum(deg_ref[...], 1.0)
    out_ref[...] = (jnp.dot(h2_ref[...], w3s_ref[...], preferred_element_type=jnp.float32)
                    + jnp.dot(agg, w3n_ref[...], preferred_element_type=jnp.float32)
                    + b3_ref[...])


def _row_spec(cols):
    return pl.BlockSpec((BR, cols), lambda i: (i, 0))


def _full_spec(*shape):
    nd = len(shape)
    return pl.BlockSpec(shape, lambda i, _n=nd: (0,) * _n)


def _hist_body(dst_hbm, cnts_hbm, dstv, counters, sem):
    wid = lax.axis_index("s") * 2 + lax.axis_index("c")
    base_e = _mo8(wid * EPW)
    dstv[pl.ds(PAD_EPW - 16, 16)] = jnp.full((16,), SINK, jnp.int32)
    pltpu.sync_copy(dst_hbm.at[pl.ds(base_e, EPW)], dstv.at[pl.ds(0, EPW)])
    z16 = jnp.zeros((16,), jnp.int32)
    for q in range(NG):
        counters[pl.ds(q * 16, 16)] = z16
    base = plsc.scan_count(z16)[0][0]

    def a1_body(g, carry):
        d = dstv[pl.ds(g * 16, 16)]
        b = lax.div(d, R)
        rank, last = plsc.scan_count(b)
        old = plsc.load_gather(counters, [b])
        plsc.store_scatter(counters, [b], old + (rank - base) + 1, mask=last)
        return carry

    lax.fori_loop(0, PAD_EPW // 16, a1_body, None)
    pltpu.sync_copy(counters, cnts_hbm.at[pl.ds(_mo8(wid * NB), NB)])


_hist = functools.partial(
    pl.kernel,
    out_type=_i32(NW * NB),
    mesh=_mesh,
    compiler_params=_sc_params,
    scratch_types=[
        pltpu.VMEM((PAD_EPW,), jnp.int32),
        pltpu.VMEM((NB,), jnp.int32),
        pltpu.SemaphoreType.DMA,
    ],
)(_hist_body)


def _align128(v):
    return lax.div(v + 127, 128) * 128


def _place_body(dst_hbm, src_hbm, cnts_hbm, packed_hbm, goff_hbm, nch_hbm,
                dstv, srcv, tmp, cnt1, staging, cntv, counters, lofft,
                gstartt, k128t, goffv, nchv, sem):
    wid = lax.axis_index("s") * 2 + lax.axis_index("c")
    base_e = _mo8(wid * EPW)
    dstv[pl.ds(PAD_EPW - 16, 16)] = jnp.full((16,), SINK, jnp.int32)
    srcv[pl.ds(PAD_EPW - 16, 16)] = jnp.zeros((16,), jnp.int32)
    pltpu.sync_copy(dst_hbm.at[pl.ds(base_e, EPW)], dstv.at[pl.ds(0, EPW)])
    pltpu.sync_copy(src_hbm.at[pl.ds(base_e, EPW)], srcv.at[pl.ds(0, EPW)])
    pltpu.sync_copy(cnts_hbm, cntv.at[pl.ds(0, NW * NB)])
    z16 = jnp.zeros((16,), jnp.int32)
    base = plsc.scan_count(z16)[0][0]

    def zc1(g, carry):
        cnt1[pl.ds(g * 16, 16)] = z16
        return carry

    lax.fori_loop(0, 50, zc1, None)

    def p1_hist(g, carry):
        d = dstv[pl.ds(g * 16, 16)]
        b = lax.div(d, R)
        loc = d - b * R
        rank, last = plsc.scan_count(loc)
        old = plsc.load_gather(cnt1, [loc])
        plsc.store_scatter(cnt1, [loc], old + (rank - base) + 1, mask=last)
        return carry

    lax.fori_loop(0, PAD_EPW // 16, p1_hist, None)

    def pfx(g, carry):
        a = cnt1[pl.ds(g * 16, 16)]
        cnt1[pl.ds(g * 16, 16)] = plsc.cumsum(a) - a + carry
        return carry + jnp.sum(a)

    lax.fori_loop(0, 50, pfx, jnp.int32(0))

    def p1_place(g, carry):
        d = dstv[pl.ds(g * 16, 16)]
        s16 = srcv[pl.ds(g * 16, 16)]
        b = lax.div(d, R)
        loc = d - b * R
        pk2 = jnp.bitwise_or(jnp.left_shift(d, 16), s16)
        rank, last = plsc.scan_count(loc)
        old = plsc.load_gather(cnt1, [loc])
        pos = old + (rank - base)
        plsc.store_scatter(cnt1, [loc], pos + 1, mask=last)
        plsc.store_scatter(tmp, [pos], pk2)
        return carry

    lax.fori_loop(0, PAD_EPW // 16, p1_place, None)

    parts = [z16] * NG
    tots = [z16] * NG
    for w2 in range(NW):
        before = jnp.int32(w2) < wid
        for q in range(NG):
            rq = cntv[pl.ds(w2 * NB + q * 16, 16)]
            aq = _align128(rq)
            parts[q] = parts[q] + jnp.where(before, aq, 0)
            tots[q] = tots[q] + aq
    carry = jnp.int32(0)
    ocarry = jnp.int32(0)
    for q in range(NG):
        goffq = plsc.cumsum(tots[q]) - tots[q] + carry
        carry = carry + jnp.sum(tots[q])
        gstartt[pl.ds(q * 16, 16)] = goffq + parts[q]
        goffv[pl.ds(q * 16, 16)] = goffq
        nchv[pl.ds(q * 16, 16)] = lax.div(tots[q], 128)
        ownq = cntv[pl.ds(wid * NB + q * 16, 16)]
        oaq = _align128(ownq)
        loffq = plsc.cumsum(oaq) - oaq + ocarry
        ocarry = ocarry + jnp.sum(oaq)
        lofft[pl.ds(q * 16, 16)] = loffq
        k128t[pl.ds(q * 16, 16)] = lax.div(oaq, 128)
        counters[pl.ds(q * 16, 16)] = loffq

    @pl.when(wid == 0)
    def _():
        pltpu.sync_copy(goffv, goff_hbm)
        pltpu.sync_copy(nchv, nch_hbm)

    def a2_body(g, carry):
        pk2 = tmp[pl.ds(g * 16, 16)]
        d = lax.shift_right_logical(pk2, 16)
        s16 = jnp.bitwise_and(pk2, 0xFFFF)
        b = lax.div(d, R)
        loc = d - b * R
        pk = jnp.bitwise_or(jnp.left_shift(loc, 16), s16)
        rank, last = plsc.scan_count(b)
        old = plsc.load_gather(counters, [b])
        pos = old + (rank - base)
        plsc.store_scatter(counters, [b], pos + 1, mask=last)
        plsc.store_scatter(staging, [pos], pk)
        return carry

    lax.fori_loop(0, PAD_EPW // 16, a2_body, None)

    sinkpk = jnp.full((16,), SINKLOC << 16, jnp.int32)
    iota = jnp.arange(16, dtype=jnp.int32)

    def hole_body(b16, carry):
        cur = counters[pl.ds(b16 * 16, 16)]
        lo = lofft[pl.ds(b16 * 16, 16)]
        k = k128t[pl.ds(b16 * 16, 16)]
        end = lo + k * 128
        for l in range(16):
            start_l = cur[l]
            end_l = end[l]
            for g in range(8):
                idx = start_l + g * 16 + iota
                plsc.store_scatter(staging, [idx], sinkpk, mask=idx < end_l)
        return carry

    lax.fori_loop(0, NG, hole_body, None)

    def out_body(b, nissued):
        lo = lofft[pl.ds(b, 16)][0]
        gs = gstartt[pl.ds(b, 16)][0]
        k = k128t[pl.ds(b, 16)][0]

        def cp_body(j, c2):
            pltpu.async_copy(
                staging.at[pl.ds(_mo8(lo + j * 128), 128)],
                packed_hbm.at[pl.ds(_mo8(gs + j * 128), 128)], sem)
            return c2

        lax.fori_loop(0, k, cp_body, None)
        return nissued + k

    nissued = lax.fori_loop(0, NB, out_body, jnp.int32(0))

    def drain_body(j, carry):
        pltpu.make_async_copy(staging.at[pl.ds(0, 128)],
                              packed_hbm.at[pl.ds(0, 128)], sem).wait()
        return carry

    lax.fori_loop(0, nissued, drain_body, None)


_place = functools.partial(
    pl.kernel,
    out_type=(_i32(GPACK), _i32(NB), _i32(NB)),
    mesh=_mesh,
    compiler_params=_sc_params,
    scratch_types=[
        pltpu.VMEM((PAD_EPW,), jnp.int32),
        pltpu.VMEM((PAD_EPW,), jnp.int32),
        pltpu.VMEM((PAD_EPW,), jnp.int32),
        pltpu.VMEM((816,), jnp.int32),
        pltpu.VMEM((STG,), jnp.int32),
        pltpu.VMEM((NW * NB + 16,), jnp.int32),
        pltpu.VMEM((NB,), jnp.int32),
        pltpu.VMEM((NB + 16,), jnp.int32),
        pltpu.VMEM((NB + 16,), jnp.int32),
        pltpu.VMEM((NB + 16,), jnp.int32),
        pltpu.VMEM((NB,), jnp.int32),
        pltpu.VMEM((NB,), jnp.int32),
        pltpu.SemaphoreType.DMA,
    ],
)(_place_body)


def _seg_reduce_body(p_hbm, packed_hbm, goff_hbm, nch_hbm, out_hbm, deg_hbm,
                     t_ref, degt, gofft, ncht, pkv, srcb, locb, rows,
                     sem_pk, sem_g, *, width, op):
    wid = lax.axis_index("s") * 2 + lax.axis_index("c")
    nv = width // 16
    z16f = jnp.zeros((16,), jnp.float32)
    ones16 = jnp.ones((16,), jnp.float32)
    pltpu.sync_copy(goff_hbm, gofft.at[pl.ds(0, NB)])
    pltpu.sync_copy(nch_hbm, ncht.at[pl.ds(0, NB)])

    def pk_dma(b0, j, buf):
        pltpu.async_copy(packed_hbm.at[pl.ds(_mo8(b0 + j * CH), CH)],
                         pkv.at[pl.ds(_mo8(buf * CH), CH)], sem_pk)

    def pk_wait(buf):
        pltpu.make_async_copy(packed_hbm.at[pl.ds(0, CH)],
                              pkv.at[pl.ds(_mo8(buf * CH), CH)], sem_pk).wait()

    def unpack(buf):
        for g in range(CH // 16):
            pkg = pkv[pl.ds(buf * CH + g * 16, 16)]
            q, off = divmod(g, 8)
            srcb[q, pl.ds(off * 16, 16)] = jnp.minimum(
                jnp.bitwise_and(pkg, 0xFFFF), N - 1)
            locb[pl.ds(g * 16, 16)] = lax.shift_right_logical(pkg, 16)

    def rmw(state, qmax):
        def qgrp(q, st):
            for sg in range(8):
                prev = st[0]
                accs = list(st[1:])
                v = locb[pl.ds(q * 128 + sg * 16, 16)]
                if op == "sum":
                    plsc.addupdate_scatter(degt, [v], ones16)
                rs = [v[l] for l in range(16)]
                for l in range(16):
                    r = rs[l]
                    rr = sg * 16 + l
                    start = r != prev
                    for k in range(nv):
                        sl = pl.ds(k * 16, 16)
                        acc_base = jnp.where(start, t_ref[r, sl], accs[k])
                        if op == "max":
                            accs[k] = jnp.maximum(acc_base, rows[q, rr, sl])
                        else:
                            accs[k] = acc_base + rows[q, rr, sl]
                    if l == 15:
                        for k in range(nv):
                            t_ref[r, pl.ds(k * 16, 16)] = accs[k]
                    else:
                        run_end = r != rs[l + 1]

                        @pl.when(run_end)
                        def _(r=r, accs_now=tuple(accs)):
                            for k in range(nv):
                                t_ref[r, pl.ds(k * 16, 16)] = accs_now[k]

                    prev = r
                st = (prev, *accs)
            return st

        return lax.fori_loop(0, qmax, qgrp, state)

    def one_bucket(half, carry):
        b = wid * 2 + half

        def zbody(r2, c2):
            for k in range(nv):
                t_ref[r2, pl.ds(k * 16, 16)] = z16f
            return c2

        lax.fori_loop(0, RT, zbody, None)
        if op == "sum":
            def zdeg(g, c2):
                degt[pl.ds(g * 16, 16)] = z16f
                return c2
            lax.fori_loop(0, 51, zdeg, None)
        base = gofft[pl.ds(b, 16)][0]
        s128 = ncht[pl.ds(b, 16)][0]
        nck = lax.div(s128 + (CQ - 1), CQ)

        @pl.when(nck > 0)
        def _():
            pltpu.sync_copy(packed_hbm.at[pl.ds(_mo8(base), CH)],
                            pkv.at[pl.ds(0, CH)])

            @pl.when(nck > 1)
            def _():
                pk_dma(base, 1, 1)

        def main_body(j, state):
            buf = lax.rem(j, 2)

            @pl.when(j > 0)
            def _():
                pk_wait(buf)

            unpack(buf)

            @pl.when(j + 2 < nck)
            def _():
                pk_dma(base, j + 2, buf)

            for q in range(CQ):
                pltpu.async_copy(p_hbm.at[srcb.at[q]], rows.at[q], sem_g)
            for q in range(CQ):
                pltpu.make_async_copy(p_hbm.at[srcb.at[q]], rows.at[q],
                                      sem_g).wait()
            return rmw(state, jnp.minimum(s128 - j * CQ, CQ))

        state0 = (jnp.int32(-1),) + tuple(
            jnp.zeros((16,), jnp.float32) for _ in range(nv))
        lax.fori_loop(0, nck, main_body, state0)
        pltpu.sync_copy(t_ref.at[pl.ds(0, R)],
                        out_hbm.at[pl.ds(_mo8(b * R), R)])
        if op == "sum":
            pltpu.sync_copy(degt.at[pl.ds(0, R)],
                            deg_hbm.at[pl.ds(_mo8(b * R), R)])
        return carry

    lax.fori_loop(0, 2, one_bucket, None)


def _make_segmax():
    def body(p_hbm, packed_hbm, goff_hbm, nch_hbm, agg_hbm,
             t_ref, gofft, ncht, pkv, srcb, locb, rows, sem_pk, sem_g):
        _seg_reduce_body(p_hbm, packed_hbm, goff_hbm, nch_hbm, agg_hbm, None,
                         t_ref, None, gofft, ncht, pkv, srcb, locb, rows,
                         sem_pk, sem_g, width=64, op="max")

    return functools.partial(
        pl.kernel,
        out_type=_f32(NPAD, 64),
        mesh=_mesh,
        compiler_params=_sc_params,
        scratch_types=[
            pltpu.VMEM((RT, 64), jnp.float32),
            pltpu.VMEM((NB + 16,), jnp.int32),
            pltpu.VMEM((NB + 16,), jnp.int32),
            pltpu.VMEM((2 * CH,), jnp.int32),
            pltpu.VMEM((CQ, 128), jnp.int32),
            pltpu.VMEM((CH + 16,), jnp.int32),
            pltpu.VMEM((CQ, 128, 64), jnp.float32),
            pltpu.SemaphoreType.DMA,
            pltpu.SemaphoreType.DMA,
        ],
    )(body)


def _make_segsum():
    def body(p_hbm, packed_hbm, goff_hbm, nch_hbm, sums_hbm, deg_hbm,
             t_ref, degt, gofft, ncht, pkv, srcb, locb, rows, sem_pk, sem_g):
        _seg_reduce_body(p_hbm, packed_hbm, goff_hbm, nch_hbm, sums_hbm,
                         deg_hbm, t_ref, degt, gofft, ncht, pkv, srcb, locb,
                         rows, sem_pk, sem_g, width=32, op="sum")

    return functools.partial(
        pl.kernel,
        out_type=(_f32(NPAD, 32), _f32(NPAD)),
        mesh=_mesh,
        compiler_params=_sc_params,
        scratch_types=[
            pltpu.VMEM((RT, 32), jnp.float32),
            pltpu.VMEM((816,), jnp.float32),
            pltpu.VMEM((NB + 16,), jnp.int32),
            pltpu.VMEM((NB + 16,), jnp.int32),
            pltpu.VMEM((2 * CH,), jnp.int32),
            pltpu.VMEM((CQ, 128), jnp.int32),
            pltpu.VMEM((CH + 16,), jnp.int32),
            pltpu.VMEM((CQ, 128, 32), jnp.float32),
            pltpu.SemaphoreType.DMA,
            pltpu.SemaphoreType.DMA,
        ],
    )(body)


_segmax = _make_segmax()
_segsum = _make_segsum()


def kernel(x, edge_index, W1_pool, b1_pool, W1_neigh, W1_self, b1,
           W2_pool, b2_pool, W2_neigh, W2_self, b2, W3_neigh, W3_self, b3):
    src = edge_index[0]
    dst = edge_index[1]
    grid = (N // BR,)

    cnts = _hist(dst)
    packed, goff, nch = _place(dst, src, cnts)

    p1, xs1 = pl.pallas_call(
        _tc1_body,
        grid=grid,
        in_specs=[_row_spec(64), _full_spec(64, 64), _full_spec(1, 64),
                  _full_spec(64, 64)],
        out_specs=[_row_spec(64), _row_spec(64)],
        out_shape=[_f32(N, 64), _f32(N, 64)],
    )(x, W1_pool, b1_pool.reshape(1, 64), W1_self)

    agg1 = _segmax(p1, packed, goff, nch)[:N]

    p2, hs2 = pl.pallas_call(
        _tc2_body,
        grid=grid,
        in_specs=[_row_spec(64), _row_spec(64), _full_spec(64, 64),
                  _full_spec(1, 64), _full_spec(64, 64), _full_spec(1, 64),
                  _full_spec(64, 32)],
        out_specs=[_row_spec(64), _row_spec(32)],
        out_shape=[_f32(N, 64), _f32(N, 32)],
    )(xs1, agg1, W1_neigh, b1.reshape(1, 64), W2_pool, b2_pool.reshape(1, 64),
      W2_self)

    agg2 = _segmax(p2, packed, goff, nch)[:N]

    h2 = pl.pallas_call(
        _tc3_body,
        grid=grid,
        in_specs=[_row_spec(32), _row_spec(64), _full_spec(64, 32),
                  _full_spec(1, 32)],
        out_specs=_row_spec(32),
        out_shape=_f32(N, 32),
    )(hs2, agg2, W2_neigh, b2.reshape(1, 32))

    sums, deg = _segsum(h2, packed, goff, nch)

    out = pl.pallas_call(
        _tc4_body,
        grid=grid,
        in_specs=[_row_spec(32), _row_spec(32), pl.BlockSpec((BR, 1), lambda i: (i, 0)),
                  _full_spec(32, 32), _full_spec(32, 32), _full_spec(1, 32)],
        out_specs=_row_spec(32),
        out_shape=_f32(N, 32),
    )(h2, sums[:N], deg[:N, None], W3_self, W3_neigh, b3.reshape(1, 32))

    return out


# speedup vs baseline: 3.0569x; 3.0569x over previous
import functools

import jax
import jax.numpy as jnp
from jax import lax
from jax.experimental import pallas as pl
from jax.experimental.pallas import tpu as pltpu
from jax.experimental.pallas import tpu_sc as plsc

N = 50000
E = 800000
NW = 32
NB = 32
R = 1568
NPAD = NB * R
EPW = E // NW
PAD_EPW = 25008
REGION = 25264
BR = 2000
SINK = NPAD - 1

_mesh = plsc.VectorSubcoreMesh(core_axis_name="c", subcore_axis_name="s")
_sc_params = pltpu.CompilerParams(needs_layout_passes=False,
                                  use_tc_tiling_on_sc=False)


def _f32(*shape):
    return jax.ShapeDtypeStruct(shape, jnp.float32)


def _i32(*shape):
    return jax.ShapeDtypeStruct(shape, jnp.int32)


def _tc1_body(x_ref, wp_ref, bp_ref, ws_ref, p1_ref, xs1_ref):
    x = x_ref[...]
    p1_ref[...] = jnp.maximum(
        jnp.dot(x, wp_ref[...], preferred_element_type=jnp.float32) + bp_ref[...],
        0.0).astype(jnp.bfloat16)
    xs1_ref[...] = jnp.dot(x, ws_ref[...], preferred_element_type=jnp.float32)


def _tc2_body(xs1_ref, agg_ref, w1n_ref, b1_ref, w2p_ref, b2p_ref, w2s_ref,
              p2_ref, hs2_ref):
    h1 = jnp.maximum(
        xs1_ref[...]
        + jnp.dot(agg_ref[...], w1n_ref[...], preferred_element_type=jnp.float32)
        + b1_ref[...], 0.0)
    p2_ref[...] = jnp.maximum(
        jnp.dot(h1, w2p_ref[...], preferred_element_type=jnp.float32) + b2p_ref[...],
        0.0).astype(jnp.bfloat16)
    hs2_ref[...] = jnp.dot(h1, w2s_ref[...], preferred_element_type=jnp.float32)


def _tc3_body(hs2_ref, agg_ref, w2n_ref, b2_ref, h2_ref):
    h2_ref[...] = (hs2_ref[...]
                   + jnp.dot(agg_ref[...], w2n_ref[...], preferred_element_type=jnp.float32)
                   + b2_ref[...])


def _tc4_body(h2_ref, sums_ref, deg_ref, w3s_ref, w3n_ref, b3_ref, out_ref):
    s = sums_ref[0] + sums_ref[1]
    d = deg_ref[0] + deg_ref[1]
    agg = s / jnp.maximum(d, 1.0)
    out_ref[...] = (jnp.dot(h2_ref[...], w3s_ref[...], preferred_element_type=jnp.float32)
                    + jnp.dot(agg, w3n_ref[...], preferred_element_type=jnp.float32)
                    + b3_ref[...])


def _row_spec(cols):
    return pl.BlockSpec((BR, cols), lambda i: (i, 0))


def _full_spec(*shape):
    nd = len(shape)
    return pl.BlockSpec(shape, lambda i, _n=nd: (0,) * _n)


def _rank16(b, base):
    c, last = plsc.scan_count(b)
    return c - base, last


def _phase_a_body(dst_hbm, src_hbm, packed_hbm, segoff_hbm, segcnt_hbm,
                  dstv, srcv, staging, counters, offabs, sem):
    wid = lax.axis_index("s") * 2 + lax.axis_index("c")
    base_e = pl.multiple_of(wid * EPW, 8)
    dstv[pl.ds(PAD_EPW - 16, 16)] = jnp.full((16,), SINK, jnp.int32)
    srcv[pl.ds(PAD_EPW - 16, 16)] = jnp.zeros((16,), jnp.int32)
    pltpu.sync_copy(dst_hbm.at[pl.ds(base_e, EPW)], dstv.at[pl.ds(0, EPW)])
    pltpu.sync_copy(src_hbm.at[pl.ds(base_e, EPW)], srcv.at[pl.ds(0, EPW)])
    z16 = jnp.zeros((16,), jnp.int32)
    counters[pl.ds(0, 16)] = z16
    counters[pl.ds(16, 16)] = z16
    base = plsc.scan_count(z16)[0][0]

    def a1_body(g, carry):
        d = dstv[pl.ds(g * 16, 16)]
        b = lax.div(d, R)
        rank, last = _rank16(b, base)
        old = plsc.load_gather(counters, [b])
        plsc.store_scatter(counters, [b], old + rank + 1, mask=last)
        return carry

    lax.fori_loop(0, PAD_EPW // 16, a1_body, None)
    pltpu.sync_copy(counters, segcnt_hbm.at[pl.ds(pl.multiple_of(wid * NB, 8), NB)])
    c0 = counters[pl.ds(0, 16)]
    c1 = counters[pl.ds(16, 16)]
    a0 = lax.div(c0 + 7, 8) * 8
    a1 = lax.div(c1 + 7, 8) * 8
    excl0 = plsc.cumsum(a0) - a0
    excl1 = plsc.cumsum(a1) - a1 + jnp.sum(a0)
    counters[pl.ds(0, 16)] = excl0
    counters[pl.ds(16, 16)] = excl1
    offabs[pl.ds(0, 16)] = excl0 + wid * REGION
    offabs[pl.ds(16, 16)] = excl1 + wid * REGION
    pltpu.sync_copy(offabs, segoff_hbm.at[pl.ds(pl.multiple_of(wid * NB, 8), NB)])

    def a2_body(g, carry):
        d = dstv[pl.ds(g * 16, 16)]
        s16 = srcv[pl.ds(g * 16, 16)]
        b = lax.div(d, R)
        loc = d - b * R
        pk = jnp.bitwise_or(jnp.left_shift(loc, 16), s16)
        rank, last = _rank16(b, base)
        old = plsc.load_gather(counters, [b])
        pos = old + rank
        plsc.store_scatter(counters, [b], pos + 1, mask=last)
        plsc.store_scatter(staging, [pos], pk)
        return carry

    lax.fori_loop(0, PAD_EPW // 16, a2_body, None)
    pltpu.sync_copy(staging, packed_hbm.at[pl.ds(pl.multiple_of(wid * REGION, 8), REGION)])


_phase_a = functools.partial(
    pl.kernel,
    out_type=(_i32(NW * REGION), _i32(NW * NB), _i32(NW * NB)),
    mesh=_mesh,
    compiler_params=_sc_params,
    scratch_types=[
        pltpu.VMEM((PAD_EPW,), jnp.int32),
        pltpu.VMEM((PAD_EPW,), jnp.int32),
        pltpu.VMEM((REGION,), jnp.int32),
        pltpu.VMEM((NB,), jnp.int32),
        pltpu.VMEM((NB,), jnp.int32),
        pltpu.SemaphoreType.DMA,
    ],
)(_phase_a_body)


CH = 256


def _segmax_body(p_hbm, packed_hbm, segoff_hbm, segcnt_hbm, agg_hbm,
                 t_ref, segoffv, segcntv, pkv, srcb, locb, rows, sem):
    wid = lax.axis_index("s") * 2 + lax.axis_index("c")
    z16 = jnp.zeros((16,), jnp.float32)

    def zbody(r, _):
        for k in range(4):
            t_ref[r, pl.ds(k * 16, 16)] = z16
        return _

    lax.fori_loop(0, R, zbody, None)
    pltpu.sync_copy(segoff_hbm, segoffv.at[pl.ds(0, NW * NB)])
    pltpu.sync_copy(segcnt_hbm, segcntv.at[pl.ds(0, NW * NB)])

    def sw_body(sw, _):
        off = segoffv[pl.ds(sw * NB + wid, 16)][0]
        cnt = segcntv[pl.ds(sw * NB + wid, 16)][0]
        nch = lax.div(cnt + (CH - 1), CH)

        def ch_body(c, _c):
            pltpu.sync_copy(packed_hbm.at[pl.ds(pl.multiple_of(off + c * CH, 8), CH)], pkv)
            for g in range(CH // 16):
                pkg = pkv[pl.ds(g * 16, 16)]
                srcb[pl.ds(g * 16, 16)] = jnp.minimum(
                    jnp.bitwise_and(pkg, 0xFFFF), N - 1)
                locb[pl.ds(g * 16, 16)] = lax.shift_right_logical(pkg, 16)
            for j in range(CH // 128):
                sl = pl.ds(j * 128, 128)
                pltpu.async_copy(p_hbm.at[srcb.at[sl]], rows.at[sl], sem).wait()
            m = jnp.minimum(cnt - c * CH, CH)
            himask = jnp.full((16,), -65536, jnp.int32)

            def e_body(e, _e):
                r = locb[pl.ds(e, 16)][0]
                for k in range(2):
                    w = plsc.bitcast(rows[e, pl.ds(k * 32, 32)], jnp.int32)
                    lo = plsc.bitcast(jnp.left_shift(w, 16), jnp.float32)
                    hi = plsc.bitcast(jnp.bitwise_and(w, himask), jnp.float32)
                    sl0 = pl.ds((2 * k) * 16, 16)
                    sl1 = pl.ds((2 * k + 1) * 16, 16)
                    t_ref[r, sl0] = jnp.maximum(t_ref[r, sl0], lo)
                    t_ref[r, sl1] = jnp.maximum(t_ref[r, sl1], hi)
                return _e

            lax.fori_loop(0, m, e_body, None)
            return _c

        lax.fori_loop(0, nch, ch_body, None)
        return _

    lax.fori_loop(0, NW, sw_body, None)
    pltpu.sync_copy(t_ref, agg_hbm.at[pl.ds(pl.multiple_of(wid * R, 8), R)])


_segmax = functools.partial(
    pl.kernel,
    out_type=_f32(NPAD, 64),
    mesh=_mesh,
    compiler_params=_sc_params,
    scratch_types=[
        pltpu.VMEM((R, 64), jnp.float32),
        pltpu.VMEM((NW * NB + 16,), jnp.int32),
        pltpu.VMEM((NW * NB + 16,), jnp.int32),
        pltpu.VMEM((CH,), jnp.int32),
        pltpu.VMEM((CH,), jnp.int32),
        pltpu.VMEM((CH + 16,), jnp.int32),
        pltpu.VMEM((CH, 64), jnp.bfloat16),
        pltpu.SemaphoreType.DMA,
    ],
)(_segmax_body)


STRIPE = NPAD // 16
FULL_CH3 = EPW // 128


def _segsum_body(h2_hbm, src_hbm, dst_hbm, sums_hbm, deg_hbm,
                 s_sh, d_sh, srcv, dstv2, rows, zblk, zvec, ones, sem):
    cid = lax.axis_index("c")
    sid = lax.axis_index("s")
    wid = sid * 2 + cid
    z16 = jnp.zeros((16,), jnp.float32)
    o16 = jnp.ones((16,), jnp.float32)

    def zb_body(r, _):
        zblk[r, pl.ds(0, 16)] = z16
        zblk[r, pl.ds(16, 16)] = z16
        return _

    lax.fori_loop(0, 112, zb_body, None)
    for g in range(25):
        zvec[pl.ds(g * 16, 16)] = z16
    for g in range(8):
        ones[pl.ds(g * 16, 16)] = o16

    def zs_body(i, _):
        pltpu.sync_copy(zblk, s_sh.at[pl.ds(pl.multiple_of(sid * STRIPE + i * 112, 8), 112)])
        return _

    lax.fori_loop(0, STRIPE // 112, zs_body, None)

    def zd_body(i, _):
        pltpu.sync_copy(zvec.at[pl.ds(0, 392)],
                        d_sh.at[pl.ds(pl.multiple_of(sid * STRIPE + i * 392, 8), 392)])
        return _

    lax.fori_loop(0, STRIPE // 392, zd_body, None)
    plsc.subcore_barrier()

    def chunk(eoff, nreal):
        if nreal < 128:
            for g in range(8):
                srcv[pl.ds(g * 16, 16)] = jnp.zeros((16,), jnp.int32)
                dstv2[0, pl.ds(g * 16, 16)] = jnp.full((16,), SINK, jnp.int32)
            pltpu.sync_copy(src_hbm.at[pl.ds(pl.multiple_of(eoff, 8), nreal)],
                            srcv.at[pl.ds(0, nreal)])
            pltpu.sync_copy(dst_hbm.at[pl.ds(pl.multiple_of(eoff, 8), nreal)],
                            dstv2.at[0, pl.ds(0, nreal)])
        else:
            pltpu.sync_copy(src_hbm.at[pl.ds(pl.multiple_of(eoff, 8), 128)], srcv)
            pltpu.sync_copy(dst_hbm.at[pl.ds(pl.multiple_of(eoff, 8), 128)], dstv2.at[0])
        pltpu.async_copy(h2_hbm.at[srcv], rows, sem).wait()
        pltpu.sync_copy(rows, s_sh.at[dstv2.at[0]], add=True)
        pltpu.sync_copy(ones, d_sh.at[dstv2.at[0]], add=True)

    def e_body(i, _):
        chunk(wid * EPW + i * 128, 128)
        return _

    lax.fori_loop(0, FULL_CH3, e_body, None)
    chunk(wid * EPW + FULL_CH3 * 128, EPW - FULL_CH3 * 128)
    plsc.subcore_barrier()
    sl = pl.ds(pl.multiple_of(sid * STRIPE, 8), STRIPE)
    pltpu.sync_copy(s_sh.at[sl], sums_hbm.at[cid, sl])
    pltpu.sync_copy(d_sh.at[sl], deg_hbm.at[cid, sl])


_segsum = functools.partial(
    pl.kernel,
    out_type=(_f32(2, NPAD, 32), _f32(2, NPAD)),
    mesh=_mesh,
    compiler_params=_sc_params,
    scratch_types=[
        pltpu.VMEM_SHARED((NPAD, 32), jnp.float32),
        pltpu.VMEM_SHARED((NPAD,), jnp.float32),
        pltpu.VMEM((128,), jnp.int32),
        pltpu.VMEM((1, 128), jnp.int32),
        pltpu.VMEM((128, 32), jnp.float32),
        pltpu.VMEM((112, 32), jnp.float32),
        pltpu.VMEM((400,), jnp.float32),
        pltpu.VMEM((128,), jnp.float32),
        pltpu.SemaphoreType.DMA,
    ],
)(_segsum_body)


def kernel(x, edge_index, W1_pool, b1_pool, W1_neigh, W1_self, b1,
           W2_pool, b2_pool, W2_neigh, W2_self, b2, W3_neigh, W3_self, b3):
    src = edge_index[0]
    dst = edge_index[1]
    grid = (N // BR,)
    perm64 = jnp.array([2 * j for j in range(16)]
                       + [2 * j + 1 for j in range(16)]
                       + [32 + 2 * j for j in range(16)]
                       + [33 + 2 * j for j in range(16)], jnp.int32)
    W1n_p = W1_neigh[perm64]
    W2n_p = W2_neigh[perm64]

    packed, segoff, segcnt = _phase_a(dst, src)

    p1, xs1 = pl.pallas_call(
        _tc1_body,
        grid=grid,
        in_specs=[_row_spec(64), _full_spec(64, 64), _full_spec(1, 64),
                  _full_spec(64, 64)],
        out_specs=[_row_spec(64), _row_spec(64)],
        out_shape=[jax.ShapeDtypeStruct((N, 64), jnp.bfloat16), _f32(N, 64)],
    )(x, W1_pool, b1_pool.reshape(1, 64), W1_self)

    agg1 = _segmax(p1, packed, segoff, segcnt)[:N]

    p2, hs2 = pl.pallas_call(
        _tc2_body,
        grid=grid,
        in_specs=[_row_spec(64), _row_spec(64), _full_spec(64, 64),
                  _full_spec(1, 64), _full_spec(64, 64), _full_spec(1, 64),
                  _full_spec(64, 32)],
        out_specs=[_row_spec(64), _row_spec(32)],
        out_shape=[jax.ShapeDtypeStruct((N, 64), jnp.bfloat16), _f32(N, 32)],
    )(xs1, agg1, W1n_p, b1.reshape(1, 64), W2_pool, b2_pool.reshape(1, 64),
      W2_self)

    agg2 = _segmax(p2, packed, segoff, segcnt)[:N]

    h2 = pl.pallas_call(
        _tc3_body,
        grid=grid,
        in_specs=[_row_spec(32), _row_spec(64), _full_spec(64, 32),
                  _full_spec(1, 32)],
        out_specs=_row_spec(32),
        out_shape=_f32(N, 32),
    )(hs2, agg2, W2n_p, b2.reshape(1, 32))

    sums, deg = _segsum(h2, src, dst)

    out = pl.pallas_call(
        _tc4_body,
        grid=grid,
        in_specs=[_row_spec(32), pl.BlockSpec((2, BR, 32), lambda i: (0, i, 0)),
                  pl.BlockSpec((2, BR, 1), lambda i: (0, i, 0)),
                  _full_spec(32, 32), _full_spec(32, 32), _full_spec(1, 32)],
        out_specs=_row_spec(32),
        out_shape=_f32(N, 32),
    )(h2, sums[:, :N], deg[:, :N, None], W3_self, W3_neigh, b3.reshape(1, 32))

    return out


# speedup vs baseline: 3.3906x; 1.1092x over previous
import functools

import jax
import jax.numpy as jnp
from jax import lax
from jax.experimental import pallas as pl
from jax.experimental.pallas import tpu as pltpu
from jax.experimental.pallas import tpu_sc as plsc

N = 50000
E = 800000
NW = 32
NB = 32
R = 1568
NPAD = NB * R
EPW = E // NW
PAD_EPW = 25008
REGION = 25264
BR = 2000
SINK = NPAD - 1

_mesh = plsc.VectorSubcoreMesh(core_axis_name="c", subcore_axis_name="s")
_sc_params = pltpu.CompilerParams(needs_layout_passes=False,
                                  use_tc_tiling_on_sc=False)


def _f32(*shape):
    return jax.ShapeDtypeStruct(shape, jnp.float32)


def _i32(*shape):
    return jax.ShapeDtypeStruct(shape, jnp.int32)


def _tc1_body(x_ref, wp_ref, bp_ref, ws_ref, p1_ref, xs1_ref):
    x = x_ref[...]
    p1_ref[...] = jnp.maximum(
        jnp.dot(x, wp_ref[...], preferred_element_type=jnp.float32) + bp_ref[...],
        0.0).astype(jnp.bfloat16)
    xs1_ref[...] = jnp.dot(x, ws_ref[...], preferred_element_type=jnp.float32)


def _tc2_body(xs1_ref, agg_ref, w1n_ref, b1_ref, w2p_ref, b2p_ref, w2s_ref,
              p2_ref, hs2_ref):
    h1 = jnp.maximum(
        xs1_ref[...]
        + jnp.dot(agg_ref[...], w1n_ref[...], preferred_element_type=jnp.float32)
        + b1_ref[...], 0.0)
    p2_ref[...] = jnp.maximum(
        jnp.dot(h1, w2p_ref[...], preferred_element_type=jnp.float32) + b2p_ref[...],
        0.0).astype(jnp.bfloat16)
    hs2_ref[...] = jnp.dot(h1, w2s_ref[...], preferred_element_type=jnp.float32)


def _tc3_body(hs2_ref, agg_ref, w2n_ref, b2_ref, h2_ref):
    h2_ref[...] = (hs2_ref[...]
                   + jnp.dot(agg_ref[...], w2n_ref[...], preferred_element_type=jnp.float32)
                   + b2_ref[...])


def _tc4_body(h2_ref, sums_ref, deg_ref, w3s_ref, w3n_ref, b3_ref, out_ref):
    s = sums_ref[0] + sums_ref[1]
    d = deg_ref[0] + deg_ref[1]
    agg = s / jnp.maximum(d, 1.0)
    out_ref[...] = (jnp.dot(h2_ref[...], w3s_ref[...], preferred_element_type=jnp.float32)
                    + jnp.dot(agg, w3n_ref[...], preferred_element_type=jnp.float32)
                    + b3_ref[...])


def _row_spec(cols):
    return pl.BlockSpec((BR, cols), lambda i: (i, 0))


def _full_spec(*shape):
    nd = len(shape)
    return pl.BlockSpec(shape, lambda i, _n=nd: (0,) * _n)


def _rank16(b, base):
    c, last = plsc.scan_count(b)
    return c - base, last


def _phase_a_body(dst_hbm, src_hbm, packed_hbm, segoff_hbm, segcnt_hbm,
                  dstv, srcv, staging, counters, offabs, sem):
    wid = lax.axis_index("s") * 2 + lax.axis_index("c")
    base_e = pl.multiple_of(wid * EPW, 8)
    dstv[pl.ds(PAD_EPW - 16, 16)] = jnp.full((16,), SINK, jnp.int32)
    srcv[pl.ds(PAD_EPW - 16, 16)] = jnp.zeros((16,), jnp.int32)
    pltpu.sync_copy(dst_hbm.at[pl.ds(base_e, EPW)], dstv.at[pl.ds(0, EPW)])
    pltpu.sync_copy(src_hbm.at[pl.ds(base_e, EPW)], srcv.at[pl.ds(0, EPW)])
    z16 = jnp.zeros((16,), jnp.int32)
    counters[pl.ds(0, 16)] = z16
    counters[pl.ds(16, 16)] = z16
    base = plsc.scan_count(z16)[0][0]

    def a1_body(g, carry):
        d = dstv[pl.ds(g * 16, 16)]
        b = lax.div(d, R)
        rank, last = _rank16(b, base)
        old = plsc.load_gather(counters, [b])
        plsc.store_scatter(counters, [b], old + rank + 1, mask=last)
        return carry

    lax.fori_loop(0, PAD_EPW // 16, a1_body, None)
    pltpu.sync_copy(counters, segcnt_hbm.at[pl.ds(pl.multiple_of(wid * NB, 8), NB)])
    c0 = counters[pl.ds(0, 16)]
    c1 = counters[pl.ds(16, 16)]
    a0 = lax.div(c0 + 7, 8) * 8
    a1 = lax.div(c1 + 7, 8) * 8
    excl0 = plsc.cumsum(a0) - a0
    excl1 = plsc.cumsum(a1) - a1 + jnp.sum(a0)
    counters[pl.ds(0, 16)] = excl0
    counters[pl.ds(16, 16)] = excl1
    offabs[pl.ds(0, 16)] = excl0 + wid * REGION
    offabs[pl.ds(16, 16)] = excl1 + wid * REGION
    pltpu.sync_copy(offabs, segoff_hbm.at[pl.ds(pl.multiple_of(wid * NB, 8), NB)])

    def a2_body(g, carry):
        d = dstv[pl.ds(g * 16, 16)]
        s16 = srcv[pl.ds(g * 16, 16)]
        b = lax.div(d, R)
        loc = d - b * R
        pk = jnp.bitwise_or(jnp.left_shift(loc, 16), s16)
        rank, last = _rank16(b, base)
        old = plsc.load_gather(counters, [b])
        pos = old + rank
        plsc.store_scatter(counters, [b], pos + 1, mask=last)
        plsc.store_scatter(staging, [pos], pk)
        return carry

    lax.fori_loop(0, PAD_EPW // 16, a2_body, None)
    pltpu.sync_copy(staging, packed_hbm.at[pl.ds(pl.multiple_of(wid * REGION, 8), REGION)])


_phase_a = functools.partial(
    pl.kernel,
    out_type=(_i32(NW * REGION), _i32(NW * NB), _i32(NW * NB)),
    mesh=_mesh,
    compiler_params=_sc_params,
    scratch_types=[
        pltpu.VMEM((PAD_EPW,), jnp.int32),
        pltpu.VMEM((PAD_EPW,), jnp.int32),
        pltpu.VMEM((REGION,), jnp.int32),
        pltpu.VMEM((NB,), jnp.int32),
        pltpu.VMEM((NB,), jnp.int32),
        pltpu.SemaphoreType.DMA,
    ],
)(_phase_a_body)


CH = 512


def _segmax_body(p_hbm, packed_hbm, segoff_hbm, segcnt_hbm, agg_hbm,
                 t_ref, segoffv, segcntv, pkv, srcb, locb, rows, sem):
    wid = lax.axis_index("s") * 2 + lax.axis_index("c")
    z16 = jnp.zeros((16,), jnp.float32)

    def zbody(r, _):
        for k in range(4):
            t_ref[r, pl.ds(k * 16, 16)] = z16
        return _

    lax.fori_loop(0, R, zbody, None)
    pltpu.sync_copy(segoff_hbm, segoffv.at[pl.ds(0, NW * NB)])
    pltpu.sync_copy(segcnt_hbm, segcntv.at[pl.ds(0, NW * NB)])

    def sw_body(sw, _):
        off = segoffv[pl.ds(sw * NB + wid, 16)][0]
        cnt = segcntv[pl.ds(sw * NB + wid, 16)][0]
        nch = lax.div(cnt + (CH - 1), CH)

        def ch_body(c, _c):
            pltpu.sync_copy(packed_hbm.at[pl.ds(pl.multiple_of(off + c * CH, 8), CH)], pkv)
            for g in range(CH // 16):
                pkg = pkv[pl.ds(g * 16, 16)]
                srcb[pl.ds(g * 16, 16)] = jnp.minimum(
                    jnp.bitwise_and(pkg, 0xFFFF), N - 1)
                locb[pl.ds(g * 16, 16)] = lax.shift_right_logical(pkg, 16)
            handles = []
            for j in range(CH // 128):
                sl = pl.ds(j * 128, 128)
                handles.append(
                    pltpu.async_copy(p_hbm.at[srcb.at[sl]], rows.at[sl], sem))
            for h in handles:
                h.wait()
            m = jnp.minimum(cnt - c * CH, CH)
            himask = jnp.full((16,), -65536, jnp.int32)

            def e_body(e, _e):
                r = locb[pl.ds(e, 16)][0]
                for k in range(2):
                    w = plsc.bitcast(rows[e, pl.ds(k * 32, 32)], jnp.int32)
                    lo = plsc.bitcast(jnp.left_shift(w, 16), jnp.float32)
                    hi = plsc.bitcast(jnp.bitwise_and(w, himask), jnp.float32)
                    sl0 = pl.ds((2 * k) * 16, 16)
                    sl1 = pl.ds((2 * k + 1) * 16, 16)
                    t_ref[r, sl0] = jnp.maximum(t_ref[r, sl0], lo)
                    t_ref[r, sl1] = jnp.maximum(t_ref[r, sl1], hi)
                return _e

            lax.fori_loop(0, m, e_body, None)
            return _c

        lax.fori_loop(0, nch, ch_body, None)
        return _

    lax.fori_loop(0, NW, sw_body, None)
    pltpu.sync_copy(t_ref, agg_hbm.at[pl.ds(pl.multiple_of(wid * R, 8), R)])


_segmax = functools.partial(
    pl.kernel,
    out_type=_f32(NPAD, 64),
    mesh=_mesh,
    compiler_params=_sc_params,
    scratch_types=[
        pltpu.VMEM((R, 64), jnp.float32),
        pltpu.VMEM((NW * NB + 16,), jnp.int32),
        pltpu.VMEM((NW * NB + 16,), jnp.int32),
        pltpu.VMEM((CH,), jnp.int32),
        pltpu.VMEM((CH,), jnp.int32),
        pltpu.VMEM((CH + 16,), jnp.int32),
        pltpu.VMEM((CH, 64), jnp.bfloat16),
        pltpu.SemaphoreType.DMA,
    ],
)(_segmax_body)


STRIPE = NPAD // 16
FULL_CH3 = EPW // 128


def _segsum_body(h2_hbm, src_hbm, dst_hbm, sums_hbm, deg_hbm,
                 s_sh, d_sh, srcv, dstv2, rows, zblk, zvec, ones, sem):
    cid = lax.axis_index("c")
    sid = lax.axis_index("s")
    wid = sid * 2 + cid
    z16 = jnp.zeros((16,), jnp.float32)
    o16 = jnp.ones((16,), jnp.float32)

    def zb_body(r, _):
        zblk[r, pl.ds(0, 16)] = z16
        zblk[r, pl.ds(16, 16)] = z16
        return _

    lax.fori_loop(0, 112, zb_body, None)
    for g in range(25):
        zvec[pl.ds(g * 16, 16)] = z16
    for g in range(8):
        ones[pl.ds(g * 16, 16)] = o16

    def zs_body(i, _):
        pltpu.sync_copy(zblk, s_sh.at[pl.ds(pl.multiple_of(sid * STRIPE + i * 112, 8), 112)])
        return _

    lax.fori_loop(0, STRIPE // 112, zs_body, None)

    def zd_body(i, _):
        pltpu.sync_copy(zvec.at[pl.ds(0, 392)],
                        d_sh.at[pl.ds(pl.multiple_of(sid * STRIPE + i * 392, 8), 392)])
        return _

    lax.fori_loop(0, STRIPE // 392, zd_body, None)
    plsc.subcore_barrier()

    def chunk(eoff, nreal):
        if nreal < 128:
            for g in range(8):
                srcv[pl.ds(g * 16, 16)] = jnp.zeros((16,), jnp.int32)
                dstv2[0, pl.ds(g * 16, 16)] = jnp.full((16,), SINK, jnp.int32)
            pltpu.sync_copy(src_hbm.at[pl.ds(pl.multiple_of(eoff, 8), nreal)],
                            srcv.at[pl.ds(0, nreal)])
            pltpu.sync_copy(dst_hbm.at[pl.ds(pl.multiple_of(eoff, 8), nreal)],
                            dstv2.at[0, pl.ds(0, nreal)])
        else:
            pltpu.sync_copy(src_hbm.at[pl.ds(pl.multiple_of(eoff, 8), 128)], srcv)
            pltpu.sync_copy(dst_hbm.at[pl.ds(pl.multiple_of(eoff, 8), 128)], dstv2.at[0])
        pltpu.async_copy(h2_hbm.at[srcv], rows, sem).wait()
        pltpu.sync_copy(rows, s_sh.at[dstv2.at[0]], add=True)
        pltpu.sync_copy(ones, d_sh.at[dstv2.at[0]], add=True)

    def e_body(i, _):
        chunk(wid * EPW + i * 128, 128)
        return _

    lax.fori_loop(0, FULL_CH3, e_body, None)
    chunk(wid * EPW + FULL_CH3 * 128, EPW - FULL_CH3 * 128)
    plsc.subcore_barrier()
    sl = pl.ds(pl.multiple_of(sid * STRIPE, 8), STRIPE)
    pltpu.sync_copy(s_sh.at[sl], sums_hbm.at[cid, sl])
    pltpu.sync_copy(d_sh.at[sl], deg_hbm.at[cid, sl])


_segsum = functools.partial(
    pl.kernel,
    out_type=(_f32(2, NPAD, 32), _f32(2, NPAD)),
    mesh=_mesh,
    compiler_params=_sc_params,
    scratch_types=[
        pltpu.VMEM_SHARED((NPAD, 32), jnp.float32),
        pltpu.VMEM_SHARED((NPAD,), jnp.float32),
        pltpu.VMEM((128,), jnp.int32),
        pltpu.VMEM((1, 128), jnp.int32),
        pltpu.VMEM((128, 32), jnp.float32),
        pltpu.VMEM((112, 32), jnp.float32),
        pltpu.VMEM((400,), jnp.float32),
        pltpu.VMEM((128,), jnp.float32),
        pltpu.SemaphoreType.DMA,
    ],
)(_segsum_body)


def kernel(x, edge_index, W1_pool, b1_pool, W1_neigh, W1_self, b1,
           W2_pool, b2_pool, W2_neigh, W2_self, b2, W3_neigh, W3_self, b3):
    src = edge_index[0]
    dst = edge_index[1]
    grid = (N // BR,)
    perm64 = jnp.array([2 * j for j in range(16)]
                       + [2 * j + 1 for j in range(16)]
                       + [32 + 2 * j for j in range(16)]
                       + [33 + 2 * j for j in range(16)], jnp.int32)
    W1n_p = W1_neigh[perm64]
    W2n_p = W2_neigh[perm64]

    packed, segoff, segcnt = _phase_a(dst, src)

    p1, xs1 = pl.pallas_call(
        _tc1_body,
        grid=grid,
        in_specs=[_row_spec(64), _full_spec(64, 64), _full_spec(1, 64),
                  _full_spec(64, 64)],
        out_specs=[_row_spec(64), _row_spec(64)],
        out_shape=[jax.ShapeDtypeStruct((N, 64), jnp.bfloat16), _f32(N, 64)],
    )(x, W1_pool, b1_pool.reshape(1, 64), W1_self)

    agg1 = _segmax(p1, packed, segoff, segcnt)[:N]

    p2, hs2 = pl.pallas_call(
        _tc2_body,
        grid=grid,
        in_specs=[_row_spec(64), _row_spec(64), _full_spec(64, 64),
                  _full_spec(1, 64), _full_spec(64, 64), _full_spec(1, 64),
                  _full_spec(64, 32)],
        out_specs=[_row_spec(64), _row_spec(32)],
        out_shape=[jax.ShapeDtypeStruct((N, 64), jnp.bfloat16), _f32(N, 32)],
    )(xs1, agg1, W1n_p, b1.reshape(1, 64), W2_pool, b2_pool.reshape(1, 64),
      W2_self)

    agg2 = _segmax(p2, packed, segoff, segcnt)[:N]

    h2 = pl.pallas_call(
        _tc3_body,
        grid=grid,
        in_specs=[_row_spec(32), _row_spec(64), _full_spec(64, 32),
                  _full_spec(1, 32)],
        out_specs=_row_spec(32),
        out_shape=_f32(N, 32),
    )(hs2, agg2, W2n_p, b2.reshape(1, 32))

    sums, deg = _segsum(h2, src, dst)

    out = pl.pallas_call(
        _tc4_body,
        grid=grid,
        in_specs=[_row_spec(32), pl.BlockSpec((2, BR, 32), lambda i: (0, i, 0)),
                  pl.BlockSpec((2, BR, 1), lambda i: (0, i, 0)),
                  _full_spec(32, 32), _full_spec(32, 32), _full_spec(1, 32)],
        out_specs=_row_spec(32),
        out_shape=_f32(N, 32),
    )(h2, sums[:, :N], deg[:, :N, None], W3_self, W3_neigh, b3.reshape(1, 32))

    return out


# speedup vs baseline: 3.7716x; 1.1124x over previous
import functools

import jax
import jax.numpy as jnp
from jax import lax
from jax.experimental import pallas as pl
from jax.experimental.pallas import tpu as pltpu
from jax.experimental.pallas import tpu_sc as plsc

N = 50000
E = 800000
NW = 32
NB = 32
R = 1568
NPAD = NB * R
EPW = E // NW
PAD_EPW = 25008
REGION = 25264
BR = 2000
SINK = NPAD - 1

_mesh = plsc.VectorSubcoreMesh(core_axis_name="c", subcore_axis_name="s")
_sc_params = pltpu.CompilerParams(needs_layout_passes=False,
                                  use_tc_tiling_on_sc=False)


def _f32(*shape):
    return jax.ShapeDtypeStruct(shape, jnp.float32)


def _i32(*shape):
    return jax.ShapeDtypeStruct(shape, jnp.int32)


def _tc1_body(x_ref, wp_ref, bp_ref, ws_ref, p1_ref, xs1_ref):
    x = x_ref[...]
    p1_ref[...] = jnp.maximum(
        jnp.dot(x, wp_ref[...], preferred_element_type=jnp.float32) + bp_ref[...],
        0.0).astype(jnp.bfloat16)
    xs1_ref[...] = jnp.dot(x, ws_ref[...], preferred_element_type=jnp.float32)


def _tc2_body(xs1_ref, agg_ref, w1n_ref, b1_ref, w2p_ref, b2p_ref, w2s_ref,
              p2_ref, hs2_ref):
    h1 = jnp.maximum(
        xs1_ref[...]
        + jnp.dot(agg_ref[...], w1n_ref[...], preferred_element_type=jnp.float32)
        + b1_ref[...], 0.0)
    p2_ref[...] = jnp.maximum(
        jnp.dot(h1, w2p_ref[...], preferred_element_type=jnp.float32) + b2p_ref[...],
        0.0).astype(jnp.bfloat16)
    hs2_ref[...] = jnp.dot(h1, w2s_ref[...], preferred_element_type=jnp.float32)


def _tc3_body(hs2_ref, agg_ref, w2n_ref, b2_ref, h2_ref):
    h2_ref[...] = (hs2_ref[...]
                   + jnp.dot(agg_ref[...], w2n_ref[...], preferred_element_type=jnp.float32)
                   + b2_ref[...])


def _tc4_body(h2_ref, sums_ref, deg_ref, w3s_ref, w3n_ref, b3_ref, out_ref):
    s = sums_ref[0] + sums_ref[1]
    d = deg_ref[0] + deg_ref[1]
    agg = s / jnp.maximum(d, 1.0)
    out_ref[...] = (jnp.dot(h2_ref[...], w3s_ref[...], preferred_element_type=jnp.float32)
                    + jnp.dot(agg, w3n_ref[...], preferred_element_type=jnp.float32)
                    + b3_ref[...])


def _row_spec(cols):
    return pl.BlockSpec((BR, cols), lambda i: (i, 0))


def _full_spec(*shape):
    nd = len(shape)
    return pl.BlockSpec(shape, lambda i, _n=nd: (0,) * _n)


def _rank16(b, base):
    c, last = plsc.scan_count(b)
    return c - base, last


def _phase_a_body(dst_hbm, src_hbm, packed_hbm, segoff_hbm, segcnt_hbm,
                  dstv, srcv, staging, counters, offabs, sem):
    wid = lax.axis_index("s") * 2 + lax.axis_index("c")
    base_e = pl.multiple_of(wid * EPW, 8)
    dstv[pl.ds(PAD_EPW - 16, 16)] = jnp.full((16,), SINK, jnp.int32)
    srcv[pl.ds(PAD_EPW - 16, 16)] = jnp.zeros((16,), jnp.int32)
    pltpu.sync_copy(dst_hbm.at[pl.ds(base_e, EPW)], dstv.at[pl.ds(0, EPW)])
    pltpu.sync_copy(src_hbm.at[pl.ds(base_e, EPW)], srcv.at[pl.ds(0, EPW)])
    z16 = jnp.zeros((16,), jnp.int32)
    counters[pl.ds(0, 16)] = z16
    counters[pl.ds(16, 16)] = z16
    base = plsc.scan_count(z16)[0][0]

    def a1_body(g, carry):
        d = dstv[pl.ds(g * 16, 16)]
        b = lax.div(d, R)
        rank, last = _rank16(b, base)
        old = plsc.load_gather(counters, [b])
        plsc.store_scatter(counters, [b], old + rank + 1, mask=last)
        return carry

    lax.fori_loop(0, PAD_EPW // 16, a1_body, None)
    pltpu.sync_copy(counters, segcnt_hbm.at[pl.ds(pl.multiple_of(wid * NB, 8), NB)])
    c0 = counters[pl.ds(0, 16)]
    c1 = counters[pl.ds(16, 16)]
    a0 = lax.div(c0 + 7, 8) * 8
    a1 = lax.div(c1 + 7, 8) * 8
    excl0 = plsc.cumsum(a0) - a0
    excl1 = plsc.cumsum(a1) - a1 + jnp.sum(a0)
    counters[pl.ds(0, 16)] = excl0
    counters[pl.ds(16, 16)] = excl1
    offabs[pl.ds(0, 16)] = excl0 + wid * REGION
    offabs[pl.ds(16, 16)] = excl1 + wid * REGION
    pltpu.sync_copy(offabs, segoff_hbm.at[pl.ds(pl.multiple_of(wid * NB, 8), NB)])

    def a2_body(g, carry):
        d = dstv[pl.ds(g * 16, 16)]
        s16 = srcv[pl.ds(g * 16, 16)]
        b = lax.div(d, R)
        loc = d - b * R
        pk = jnp.bitwise_or(jnp.left_shift(loc, 16), s16)
        rank, last = _rank16(b, base)
        old = plsc.load_gather(counters, [b])
        pos = old + rank
        plsc.store_scatter(counters, [b], pos + 1, mask=last)
        plsc.store_scatter(staging, [pos], pk)
        return carry

    lax.fori_loop(0, PAD_EPW // 16, a2_body, None)
    pltpu.sync_copy(staging, packed_hbm.at[pl.ds(pl.multiple_of(wid * REGION, 8), REGION)])


_phase_a = functools.partial(
    pl.kernel,
    out_type=(_i32(NW * REGION), _i32(NW * NB), _i32(NW * NB)),
    mesh=_mesh,
    compiler_params=_sc_params,
    scratch_types=[
        pltpu.VMEM((PAD_EPW,), jnp.int32),
        pltpu.VMEM((PAD_EPW,), jnp.int32),
        pltpu.VMEM((REGION,), jnp.int32),
        pltpu.VMEM((NB,), jnp.int32),
        pltpu.VMEM((NB,), jnp.int32),
        pltpu.SemaphoreType.DMA,
    ],
)(_phase_a_body)


CH = 512


def _segmax_body(p_hbm, packed_hbm, segoff_hbm, segcnt_hbm, agg_hbm,
                 t_ref, segoffv, segcntv, pkv, srcb, locb, rows, sem):
    wid = lax.axis_index("s") * 2 + lax.axis_index("c")
    z16 = jnp.zeros((16,), jnp.float32)

    def zbody(r, _):
        for k in range(4):
            t_ref[r, pl.ds(k * 16, 16)] = z16
        return _

    lax.fori_loop(0, R, zbody, None)
    pltpu.sync_copy(segoff_hbm, segoffv.at[pl.ds(0, NW * NB)])
    pltpu.sync_copy(segcnt_hbm, segcntv.at[pl.ds(0, NW * NB)])

    def sw_body(sw, _):
        off = segoffv[pl.ds(sw * NB + wid, 16)][0]
        cnt = segcntv[pl.ds(sw * NB + wid, 16)][0]
        nch = lax.div(cnt + (CH - 1), CH)

        def ch_body(c, _c):
            pltpu.sync_copy(packed_hbm.at[pl.ds(pl.multiple_of(off + c * CH, 8), CH)], pkv)
            for g in range(CH // 16):
                pkg = pkv[pl.ds(g * 16, 16)]
                srcb[pl.ds(g * 16, 16)] = jnp.minimum(
                    jnp.bitwise_and(pkg, 0xFFFF), N - 1)
                locb[pl.ds(g * 16, 16)] = lax.shift_right_logical(pkg, 16)
            handles = []
            for j in range(CH // 128):
                sl = pl.ds(j * 128, 128)
                handles.append(
                    pltpu.async_copy(p_hbm.at[srcb.at[sl]], rows.at[sl], sem))
            for h in handles:
                h.wait()
            m = jnp.minimum(cnt - c * CH, CH)
            himask = jnp.full((16,), -65536, jnp.int32)

            def e_body(e, _e):
                r = locb[pl.ds(e, 16)][0]
                for k in range(2):
                    w = plsc.bitcast(rows[e, pl.ds(k * 32, 32)], jnp.int32)
                    lo = plsc.bitcast(jnp.left_shift(w, 16), jnp.float32)
                    hi = plsc.bitcast(jnp.bitwise_and(w, himask), jnp.float32)
                    sl0 = pl.ds((2 * k) * 16, 16)
                    sl1 = pl.ds((2 * k + 1) * 16, 16)
                    t_ref[r, sl0] = jnp.maximum(t_ref[r, sl0], lo)
                    t_ref[r, sl1] = jnp.maximum(t_ref[r, sl1], hi)
                return _e

            lax.fori_loop(0, m, e_body, None)
            return _c

        lax.fori_loop(0, nch, ch_body, None)
        return _

    lax.fori_loop(0, NW, sw_body, None)
    pltpu.sync_copy(t_ref, agg_hbm.at[pl.ds(pl.multiple_of(wid * R, 8), R)])


_segmax = functools.partial(
    pl.kernel,
    out_type=_f32(NPAD, 64),
    mesh=_mesh,
    compiler_params=_sc_params,
    scratch_types=[
        pltpu.VMEM((R, 64), jnp.float32),
        pltpu.VMEM((NW * NB + 16,), jnp.int32),
        pltpu.VMEM((NW * NB + 16,), jnp.int32),
        pltpu.VMEM((CH,), jnp.int32),
        pltpu.VMEM((CH,), jnp.int32),
        pltpu.VMEM((CH + 16,), jnp.int32),
        pltpu.VMEM((CH, 64), jnp.bfloat16),
        pltpu.SemaphoreType.DMA,
    ],
)(_segmax_body)


STRIPE = NPAD // 16
FULL_CH3 = EPW // 512


def _segsum_body(h2_hbm, src_hbm, dst_hbm, sums_hbm, deg_hbm,
                 s_sh, d_sh, srcv, dstf, dstv2, rows, zblk, zvec, ones, sem):
    cid = lax.axis_index("c")
    sid = lax.axis_index("s")
    wid = sid * 2 + cid
    z16 = jnp.zeros((16,), jnp.float32)
    o16 = jnp.ones((16,), jnp.float32)

    def zb_body(r, _):
        zblk[r, pl.ds(0, 16)] = z16
        zblk[r, pl.ds(16, 16)] = z16
        return _

    lax.fori_loop(0, 112, zb_body, None)
    for g in range(25):
        zvec[pl.ds(g * 16, 16)] = z16
    for g in range(8):
        ones[pl.ds(g * 16, 16)] = o16

    def zs_body(i, _):
        pltpu.sync_copy(zblk, s_sh.at[pl.ds(pl.multiple_of(sid * STRIPE + i * 112, 8), 112)])
        return _

    lax.fori_loop(0, STRIPE // 112, zs_body, None)

    def zd_body(i, _):
        pltpu.sync_copy(zvec.at[pl.ds(0, 392)],
                        d_sh.at[pl.ds(pl.multiple_of(sid * STRIPE + i * 392, 8), 392)])
        return _

    lax.fori_loop(0, STRIPE // 392, zd_body, None)
    plsc.subcore_barrier()

    def chunk(eoff, nreal):
        if nreal < 512:
            for g in range(32):
                srcv[pl.ds(g * 16, 16)] = jnp.zeros((16,), jnp.int32)
                dstf[pl.ds(g * 16, 16)] = jnp.full((16,), SINK, jnp.int32)
        hs = [pltpu.async_copy(
                  src_hbm.at[pl.ds(pl.multiple_of(eoff, 8), nreal)],
                  srcv.at[pl.ds(0, nreal)], sem),
              pltpu.async_copy(
                  dst_hbm.at[pl.ds(pl.multiple_of(eoff, 8), nreal)],
                  dstf.at[pl.ds(0, nreal)], sem)]
        for h in hs:
            h.wait()
        for q in range(4):
            for g in range(8):
                dstv2[q, pl.ds(g * 16, 16)] = dstf[pl.ds(q * 128 + g * 16, 16)]
        hs = [pltpu.async_copy(h2_hbm.at[srcv.at[pl.ds(q * 128, 128)]],
                               rows.at[pl.ds(q * 128, 128)], sem)
              for q in range(4)]
        for h in hs:
            h.wait()
        hs = []
        for q in range(4):
            hs.append(pltpu.async_copy(rows.at[pl.ds(q * 128, 128)],
                                       s_sh.at[dstv2.at[q]], add=True,
                                       sem=sem))
            hs.append(pltpu.async_copy(ones, d_sh.at[dstv2.at[q]], add=True,
                                       sem=sem))
        for h in hs:
            h.wait()

    def e_body(i, _):
        chunk(wid * EPW + i * 512, 512)
        return _

    lax.fori_loop(0, FULL_CH3, e_body, None)
    chunk(wid * EPW + FULL_CH3 * 512, EPW - FULL_CH3 * 512)
    plsc.subcore_barrier()
    sl = pl.ds(pl.multiple_of(sid * STRIPE, 8), STRIPE)
    pltpu.sync_copy(s_sh.at[sl], sums_hbm.at[cid, sl])
    pltpu.sync_copy(d_sh.at[sl], deg_hbm.at[cid, sl])


_segsum = functools.partial(
    pl.kernel,
    out_type=(_f32(2, NPAD, 32), _f32(2, NPAD)),
    mesh=_mesh,
    compiler_params=_sc_params,
    scratch_types=[
        pltpu.VMEM_SHARED((NPAD, 32), jnp.float32),
        pltpu.VMEM_SHARED((NPAD,), jnp.float32),
        pltpu.VMEM((512,), jnp.int32),
        pltpu.VMEM((512,), jnp.int32),
        pltpu.VMEM((4, 128), jnp.int32),
        pltpu.VMEM((512, 32), jnp.float32),
        pltpu.VMEM((112, 32), jnp.float32),
        pltpu.VMEM((400,), jnp.float32),
        pltpu.VMEM((128,), jnp.float32),
        pltpu.SemaphoreType.DMA,
    ],
)(_segsum_body)


def kernel(x, edge_index, W1_pool, b1_pool, W1_neigh, W1_self, b1,
           W2_pool, b2_pool, W2_neigh, W2_self, b2, W3_neigh, W3_self, b3):
    src = edge_index[0]
    dst = edge_index[1]
    grid = (N // BR,)
    perm64 = jnp.array([2 * j for j in range(16)]
                       + [2 * j + 1 for j in range(16)]
                       + [32 + 2 * j for j in range(16)]
                       + [33 + 2 * j for j in range(16)], jnp.int32)
    W1n_p = W1_neigh[perm64]
    W2n_p = W2_neigh[perm64]

    packed, segoff, segcnt = _phase_a(dst, src)

    p1, xs1 = pl.pallas_call(
        _tc1_body,
        grid=grid,
        in_specs=[_row_spec(64), _full_spec(64, 64), _full_spec(1, 64),
                  _full_spec(64, 64)],
        out_specs=[_row_spec(64), _row_spec(64)],
        out_shape=[jax.ShapeDtypeStruct((N, 64), jnp.bfloat16), _f32(N, 64)],
    )(x, W1_pool, b1_pool.reshape(1, 64), W1_self)

    agg1 = _segmax(p1, packed, segoff, segcnt)[:N]

    p2, hs2 = pl.pallas_call(
        _tc2_body,
        grid=grid,
        in_specs=[_row_spec(64), _row_spec(64), _full_spec(64, 64),
                  _full_spec(1, 64), _full_spec(64, 64), _full_spec(1, 64),
                  _full_spec(64, 32)],
        out_specs=[_row_spec(64), _row_spec(32)],
        out_shape=[jax.ShapeDtypeStruct((N, 64), jnp.bfloat16), _f32(N, 32)],
    )(xs1, agg1, W1n_p, b1.reshape(1, 64), W2_pool, b2_pool.reshape(1, 64),
      W2_self)

    agg2 = _segmax(p2, packed, segoff, segcnt)[:N]

    h2 = pl.pallas_call(
        _tc3_body,
        grid=grid,
        in_specs=[_row_spec(32), _row_spec(64), _full_spec(64, 32),
                  _full_spec(1, 32)],
        out_specs=_row_spec(32),
        out_shape=_f32(N, 32),
    )(hs2, agg2, W2n_p, b2.reshape(1, 32))

    sums, deg = _segsum(h2, src, dst)

    out = pl.pallas_call(
        _tc4_body,
        grid=grid,
        in_specs=[_row_spec(32), pl.BlockSpec((2, BR, 32), lambda i: (0, i, 0)),
                  pl.BlockSpec((2, BR, 1), lambda i: (0, i, 0)),
                  _full_spec(32, 32), _full_spec(32, 32), _full_spec(1, 32)],
        out_specs=_row_spec(32),
        out_shape=_f32(N, 32),
    )(h2, sums[:, :N], deg[:, :N, None], W3_self, W3_neigh, b3.reshape(1, 32))

    return out
